# Initial kernel scaffold; baseline (speedup 1.0000x reference)
import functools

import jax
import jax.numpy as jnp
from jax import lax
from jax.experimental import pallas as pl
from jax.experimental.pallas import tpu as pltpu
from jax.experimental.pallas import tpu_sc as plsc

N = 100000
E = 6400000
NPAD = 102400
ROWS = 800
LANES = 128

NC = 2
NS = 16
NW = NC * NS
PER_W = E // NW
CHUNK = 8000
NCHUNK = PER_W // CHUNK


def _tc_att_loss(r_ref, x_ref, att_ref, loss_ref):
    x = x_ref[...]
    att = jax.nn.sigmoid(x)
    att_ref[...] = att
    r = r_ref[0]
    row = lax.broadcasted_iota(jnp.int32, (ROWS, LANES), 0)
    col = lax.broadcasted_iota(jnp.int32, (ROWS, LANES), 1)
    valid = (row * LANES + col) < N
    term = (att * jnp.log(att / r + 1e-6)
            + (1.0 - att) * jnp.log((1.0 - att) / (1.0 - r + 1e-6) + 1e-6))
    loss_ref[0] = jnp.sum(jnp.where(valid, term, 0.0)) / N


_mesh = plsc.VectorSubcoreMesh(core_axis_name="c", subcore_axis_name="s")


@functools.partial(
    pl.kernel,
    mesh=_mesh,
    out_type=jax.ShapeDtypeStruct((E,), jnp.float32),
    scratch_types=[
        pltpu.VMEM((NPAD,), jnp.float32),
        pltpu.VMEM((CHUNK,), jnp.int32),
        pltpu.VMEM((CHUNK,), jnp.int32),
        pltpu.VMEM((CHUNK,), jnp.float32),
    ],
)
def _sc_edge_att(att_hbm, ei_hbm, out_hbm, tab, sidx, didx, obuf):
    wid = lax.axis_index("s") * NC + lax.axis_index("c")
    base = wid * PER_W
    pltpu.sync_copy(att_hbm, tab)

    def chunk_body(c, carry):
        off = pl.multiple_of(base + c * CHUNK, 8)
        pltpu.sync_copy(ei_hbm.at[0, pl.ds(off, CHUNK)], sidx)
        pltpu.sync_copy(ei_hbm.at[1, pl.ds(off, CHUNK)], didx)

        def body(i, carry2):
            s = sidx[pl.ds(i * 16, 16)]
            d = didx[pl.ds(i * 16, 16)]
            a = plsc.load_gather(tab, [s])
            b = plsc.load_gather(tab, [d])
            obuf[pl.ds(i * 16, 16)] = a * b
            return carry2

        lax.fori_loop(0, CHUNK // 16, body, 0)
        pltpu.sync_copy(obuf, out_hbm.at[pl.ds(off, CHUNK)])
        return carry

    lax.fori_loop(0, NCHUNK, chunk_body, 0)


def kernel(att_log_logits, edge_index, epoch):
    r = jnp.maximum(0.9 - (epoch // 10).astype(jnp.float32) * 0.1, 0.7)
    x = jnp.pad(att_log_logits.reshape(-1), (0, NPAD - N)).reshape(ROWS, LANES)

    att2d, loss = pl.pallas_call(
        _tc_att_loss,
        in_specs=[
            pl.BlockSpec(memory_space=pltpu.SMEM),
            pl.BlockSpec(memory_space=pltpu.VMEM),
        ],
        out_specs=[
            pl.BlockSpec(memory_space=pltpu.VMEM),
            pl.BlockSpec(memory_space=pltpu.SMEM),
        ],
        out_shape=[
            jax.ShapeDtypeStruct((ROWS, LANES), jnp.float32),
            jax.ShapeDtypeStruct((1,), jnp.float32),
        ],
    )(r.reshape(1), x)

    edge_att = _sc_edge_att(att2d.reshape(NPAD), edge_index)
    return edge_att.reshape(E, 1), loss[0]


# speedup vs baseline: 587.5959x; 587.5959x over previous
import functools

import jax
import jax.numpy as jnp
from jax import lax
from jax.experimental import pallas as pl
from jax.experimental.pallas import tpu as pltpu
from jax.experimental.pallas import tpu_sc as plsc

N = 100000
E = 6400000
NPAD = 102400
ROWS = 800
LANES = 128

NC = 2
NS = 16
NW = NC * NS
PER_W = E // NW
CHUNK = 8000
NCHUNK = PER_W // CHUNK


def _tc_att_loss(r_ref, x_ref, att_ref, loss_ref):
    x = x_ref[...]
    att = jax.nn.sigmoid(x)
    att_ref[...] = att
    r = r_ref[0]
    row = lax.broadcasted_iota(jnp.int32, (ROWS, LANES), 0)
    col = lax.broadcasted_iota(jnp.int32, (ROWS, LANES), 1)
    valid = (row * LANES + col) < N
    term = (att * jnp.log(att / r + 1e-6)
            + (1.0 - att) * jnp.log((1.0 - att) / (1.0 - r + 1e-6) + 1e-6))
    loss_ref[0] = jnp.sum(jnp.where(valid, term, 0.0)) / N


_mesh = plsc.VectorSubcoreMesh(core_axis_name="c", subcore_axis_name="s")


@functools.partial(
    pl.kernel,
    mesh=_mesh,
    out_type=jax.ShapeDtypeStruct((E,), jnp.float32),
    scratch_types=[
        pltpu.VMEM((NPAD,), jnp.float32),
        pltpu.VMEM((CHUNK,), jnp.int32),
        pltpu.VMEM((CHUNK,), jnp.int32),
        pltpu.VMEM((CHUNK,), jnp.float32),
    ],
    compiler_params=pltpu.CompilerParams(needs_layout_passes=False),
)
def _sc_edge_att(att_hbm, ei_hbm, out_hbm, tab, sidx, didx, obuf):
    wid = lax.axis_index("s") * NC + lax.axis_index("c")
    base = wid * PER_W
    pltpu.sync_copy(att_hbm, tab)

    def chunk_body(c, carry):
        off = pl.multiple_of(base + c * CHUNK, 8)
        pltpu.sync_copy(ei_hbm.at[pl.ds(off, CHUNK)], sidx)
        pltpu.sync_copy(ei_hbm.at[pl.ds(E + off, CHUNK)], didx)

        def body(i, carry2):
            s = sidx[pl.ds(i * 16, 16)]
            d = didx[pl.ds(i * 16, 16)]
            a = plsc.load_gather(tab, [s])
            b = plsc.load_gather(tab, [d])
            obuf[pl.ds(i * 16, 16)] = a * b
            return carry2

        lax.fori_loop(0, CHUNK // 16, body, 0)
        pltpu.sync_copy(obuf, out_hbm.at[pl.ds(off, CHUNK)])
        return carry

    lax.fori_loop(0, NCHUNK, chunk_body, 0)


def kernel(att_log_logits, edge_index, epoch):
    r = jnp.maximum(0.9 - (epoch // 10).astype(jnp.float32) * 0.1, 0.7)
    x = jnp.pad(att_log_logits.reshape(-1), (0, NPAD - N)).reshape(ROWS, LANES)

    att2d, loss = pl.pallas_call(
        _tc_att_loss,
        in_specs=[
            pl.BlockSpec(memory_space=pltpu.SMEM),
            pl.BlockSpec(memory_space=pltpu.VMEM),
        ],
        out_specs=[
            pl.BlockSpec(memory_space=pltpu.VMEM),
            pl.BlockSpec(memory_space=pltpu.SMEM),
        ],
        out_shape=[
            jax.ShapeDtypeStruct((ROWS, LANES), jnp.float32),
            jax.ShapeDtypeStruct((1,), jnp.float32),
        ],
    )(r.reshape(1), x)

    edge_att = _sc_edge_att(att2d.reshape(NPAD), edge_index.reshape(2 * E))
    return edge_att.reshape(E, 1), loss[0]


# speedup vs baseline: 1038.8869x; 1.7680x over previous
import functools

import jax
import jax.numpy as jnp
from jax import lax
from jax.experimental import pallas as pl
from jax.experimental.pallas import tpu as pltpu
from jax.experimental.pallas import tpu_sc as plsc

N = 100000
E = 6400000
NPAD = 102400
ROWS = 800
LANES = 128

NC = 2
NS = 16
NW = NC * NS
CHUNK = 2048
NCHT = E // CHUNK
BASE_CH = NCHT // NW
EXTRA = NCHT % NW
MAXCH = BASE_CH + 1


def _tc_att_loss(r_ref, x_ref, att_ref, loss_ref):
    x = x_ref[...]
    att = jax.nn.sigmoid(x)
    att_ref[...] = att
    r = r_ref[0]
    row = lax.broadcasted_iota(jnp.int32, (ROWS, LANES), 0)
    col = lax.broadcasted_iota(jnp.int32, (ROWS, LANES), 1)
    valid = (row * LANES + col) < N
    term = (att * jnp.log(att / r + 1e-6)
            + (1.0 - att) * jnp.log((1.0 - att) / (1.0 - r + 1e-6) + 1e-6))
    loss_ref[0] = jnp.sum(jnp.where(valid, term, 0.0)) / N


_mesh = plsc.VectorSubcoreMesh(core_axis_name="c", subcore_axis_name="s")


@functools.partial(
    pl.kernel,
    mesh=_mesh,
    out_type=jax.ShapeDtypeStruct((E,), jnp.float32),
    scratch_types=[
        pltpu.VMEM((NPAD,), jnp.float32),
        pltpu.VMEM((2, 2, CHUNK), jnp.int32),
        pltpu.VMEM((2, CHUNK), jnp.float32),
        pltpu.SemaphoreType.DMA,
        pltpu.SemaphoreType.DMA,
    ],
    compiler_params=pltpu.CompilerParams(needs_layout_passes=False),
)
def _sc_edge_att(att_hbm, ei_hbm, out_hbm, tab, ibuf, obuf, insem, outsem):
    w = lax.axis_index("s") * NC + lax.axis_index("c")
    nch = BASE_CH + (w < EXTRA).astype(jnp.int32)
    pltpu.sync_copy(att_hbm, tab)

    def in_off(c):
        return pl.multiple_of((c * NW + w) * CHUNK, CHUNK)

    pltpu.async_copy(ei_hbm.at[:, pl.ds(in_off(0), CHUNK)], ibuf.at[0], insem)

    def pair_body(g, carry):
        for b in (0, 1):
            c = g * 2 + b

            @pl.when(c < nch)
            def _():
                off = in_off(c)
                pltpu.make_async_copy(
                    ei_hbm.at[:, pl.ds(off, CHUNK)], ibuf.at[b], insem
                ).wait()

                @pl.when(c + 1 < nch)
                def _():
                    pltpu.async_copy(
                        ei_hbm.at[:, pl.ds(in_off(c + 1), CHUNK)],
                        ibuf.at[1 - b], insem,
                    )

                @pl.when(c >= 2)
                def _():
                    pltpu.make_async_copy(
                        obuf.at[b], out_hbm.at[pl.ds(off, CHUNK)], outsem
                    ).wait()

                @plsc.parallel_loop(0, CHUNK // 16, 1, unroll=8)
                def _(j):
                    s = ibuf[b, 0, pl.ds(j * 16, 16)]
                    d = ibuf[b, 1, pl.ds(j * 16, 16)]
                    obuf[b, pl.ds(j * 16, 16)] = (
                        plsc.load_gather(tab, [s]) * plsc.load_gather(tab, [d])
                    )

                pltpu.async_copy(
                    obuf.at[b], out_hbm.at[pl.ds(off, CHUNK)], outsem
                )

        return carry

    lax.fori_loop(0, (MAXCH + 1) // 2, pair_body, 0)

    for b in (0, 1):
        pltpu.make_async_copy(
            obuf.at[b], out_hbm.at[pl.ds(in_off(0), CHUNK)], outsem
        ).wait()


def kernel(att_log_logits, edge_index, epoch):
    r = jnp.maximum(0.9 - (epoch // 10).astype(jnp.float32) * 0.1, 0.7)
    x = jnp.pad(att_log_logits.reshape(-1), (0, NPAD - N)).reshape(ROWS, LANES)

    att2d, loss = pl.pallas_call(
        _tc_att_loss,
        in_specs=[
            pl.BlockSpec(memory_space=pltpu.SMEM),
            pl.BlockSpec(memory_space=pltpu.VMEM),
        ],
        out_specs=[
            pl.BlockSpec(memory_space=pltpu.VMEM),
            pl.BlockSpec(memory_space=pltpu.SMEM),
        ],
        out_shape=[
            jax.ShapeDtypeStruct((ROWS, LANES), jnp.float32),
            jax.ShapeDtypeStruct((1,), jnp.float32),
        ],
    )(r.reshape(1), x)

    edge_att = _sc_edge_att(att2d.reshape(NPAD), edge_index)
    return edge_att.reshape(E, 1), loss[0]


# speedup vs baseline: 1373.0231x; 1.3216x over previous
import functools

import jax
import jax.numpy as jnp
from jax import lax
from jax.experimental import pallas as pl
from jax.experimental.pallas import tpu as pltpu
from jax.experimental.pallas import tpu_sc as plsc

N = 100000
E = 6400000
NPAD = 102400
ROWS = 800
LANES = 128

NC = 2
NS = 16
NW = NC * NS
BLK = 2048
NBLK = E // BLK
BASE_BLK = NBLK // NW
EXTRA = NBLK % NW
CC = 2 * BLK
NCOPIES = 48


def _tc_att_loss(r_ref, x_ref, att_ref, loss_ref):
    x = x_ref[...]
    att = jax.nn.sigmoid(x)
    att_ref[...] = att
    r = r_ref[0]
    row = lax.broadcasted_iota(jnp.int32, (ROWS, LANES), 0)
    col = lax.broadcasted_iota(jnp.int32, (ROWS, LANES), 1)
    valid = (row * LANES + col) < N
    term = (att * jnp.log(att / r + 1e-6)
            + (1.0 - att) * jnp.log((1.0 - att) / (1.0 - r + 1e-6) + 1e-6))
    loss_ref[0] = jnp.sum(jnp.where(valid, term, 0.0)) / N


_mesh = plsc.VectorSubcoreMesh(core_axis_name="c", subcore_axis_name="s")


@functools.partial(
    pl.kernel,
    mesh=_mesh,
    out_type=jax.ShapeDtypeStruct((E,), jnp.float32),
    scratch_types=[
        pltpu.VMEM((NPAD,), jnp.float32),
        pltpu.VMEM((2, 2, CC), jnp.int32),
        pltpu.VMEM((2, CC), jnp.float32),
        pltpu.SemaphoreType.DMA,
        pltpu.SemaphoreType.DMA,
        pltpu.SemaphoreType.DMA,
    ],
    compiler_params=pltpu.CompilerParams(needs_layout_passes=False),
)
def _sc_edge_att(att_hbm, ei_hbm, out_hbm, tab, ibuf, obuf, insem, outsem, tabsem):
    w = lax.axis_index("s") * NC + lax.axis_index("c")
    base = (w * BASE_BLK + jnp.minimum(w, EXTRA)) * BLK

    def in_off(c):
        return pl.multiple_of(base + c * CC, BLK)

    pltpu.async_copy(att_hbm, tab, tabsem)
    pltpu.async_copy(ei_hbm.at[:, pl.ds(in_off(0), CC)], ibuf.at[0], insem)
    pltpu.make_async_copy(att_hbm, tab, tabsem).wait()

    def pair_body(g, carry):
        for b in (0, 1):
            c = g * 2 + b
            off = in_off(c)
            pltpu.make_async_copy(
                ei_hbm.at[:, pl.ds(off, CC)], ibuf.at[b], insem
            ).wait()

            @pl.when(c + 1 < NCOPIES)
            def _():
                pltpu.async_copy(
                    ei_hbm.at[:, pl.ds(in_off(c + 1), CC)], ibuf.at[1 - b], insem
                )

            @pl.when(c >= 2)
            def _():
                pltpu.make_async_copy(
                    obuf.at[b], out_hbm.at[pl.ds(off, CC)], outsem
                ).wait()

            @plsc.parallel_loop(0, CC // 16, 1, unroll=8)
            def _(j):
                s = ibuf[b, 0, pl.ds(j * 16, 16)]
                d = ibuf[b, 1, pl.ds(j * 16, 16)]
                obuf[b, pl.ds(j * 16, 16)] = (
                    plsc.load_gather(tab, [s]) * plsc.load_gather(tab, [d])
                )

            pltpu.async_copy(obuf.at[b], out_hbm.at[pl.ds(off, CC)], outsem)
        return carry

    lax.fori_loop(0, NCOPIES // 2, pair_body, 0)

    for b in (0, 1):
        pltpu.make_async_copy(
            obuf.at[b], out_hbm.at[pl.ds(in_off(0), CC)], outsem
        ).wait()

    ntail = 1 + (w < EXTRA).astype(jnp.int32)

    def tail_body(t, carry):
        toff = pl.multiple_of(base + NCOPIES * CC + t * BLK, BLK)
        pltpu.sync_copy(ei_hbm.at[:, pl.ds(toff, BLK)], ibuf.at[0, :, pl.ds(0, BLK)])

        @plsc.parallel_loop(0, BLK // 16, 1, unroll=8)
        def _(j):
            s = ibuf[0, 0, pl.ds(j * 16, 16)]
            d = ibuf[0, 1, pl.ds(j * 16, 16)]
            obuf[0, pl.ds(j * 16, 16)] = (
                plsc.load_gather(tab, [s]) * plsc.load_gather(tab, [d])
            )

        pltpu.sync_copy(obuf.at[0, pl.ds(0, BLK)], out_hbm.at[pl.ds(toff, BLK)])
        return carry

    lax.fori_loop(0, ntail, tail_body, 0)


def kernel(att_log_logits, edge_index, epoch):
    r = jnp.maximum(0.9 - (epoch // 10).astype(jnp.float32) * 0.1, 0.7)
    x = jnp.pad(att_log_logits.reshape(-1), (0, NPAD - N)).reshape(ROWS, LANES)

    att2d, loss = pl.pallas_call(
        _tc_att_loss,
        in_specs=[
            pl.BlockSpec(memory_space=pltpu.SMEM),
            pl.BlockSpec(memory_space=pltpu.VMEM),
        ],
        out_specs=[
            pl.BlockSpec(memory_space=pltpu.VMEM),
            pl.BlockSpec(memory_space=pltpu.SMEM),
        ],
        out_shape=[
            jax.ShapeDtypeStruct((ROWS, LANES), jnp.float32),
            jax.ShapeDtypeStruct((1,), jnp.float32),
        ],
    )(r.reshape(1), x)

    edge_att = _sc_edge_att(att2d.reshape(NPAD), edge_index)
    return edge_att.reshape(E, 1), loss[0]


# speedup vs baseline: 1650.8664x; 1.2024x over previous
import functools

import jax
import jax.numpy as jnp
from jax import lax
from jax.experimental import pallas as pl
from jax.experimental.pallas import tpu as pltpu
from jax.experimental.pallas import tpu_sc as plsc

N = 100000
E = 6400000
NPAD = 102400
ROWS = 800
LANES = 128

NC = 2
NS = 16
NW = NC * NS
BLK = 2048
NBLK = E // BLK
BASE_BLK = NBLK // NW
EXTRA = NBLK % NW
CC = BLK
NCOPIES = 96
NBUF = 4


def _tc_att_loss(r_ref, x_ref, att_ref, loss_ref):
    x = x_ref[...]
    att = jax.nn.sigmoid(x)
    att_ref[...] = att
    r = r_ref[0]
    row = lax.broadcasted_iota(jnp.int32, (ROWS, LANES), 0)
    col = lax.broadcasted_iota(jnp.int32, (ROWS, LANES), 1)
    valid = (row * LANES + col) < N
    term = (att * jnp.log(att / r + 1e-6)
            + (1.0 - att) * jnp.log((1.0 - att) / (1.0 - r + 1e-6) + 1e-6))
    loss_ref[0] = jnp.sum(jnp.where(valid, term, 0.0)) / N


_mesh = plsc.VectorSubcoreMesh(core_axis_name="c", subcore_axis_name="s")


@functools.partial(
    pl.kernel,
    mesh=_mesh,
    out_type=jax.ShapeDtypeStruct((E,), jnp.float32),
    scratch_types=[
        pltpu.VMEM((NPAD,), jnp.float32),
        pltpu.VMEM((NBUF, 2, CC), jnp.int32),
        pltpu.VMEM((NBUF, CC), jnp.float32),
        pltpu.SemaphoreType.DMA,
        pltpu.SemaphoreType.DMA,
        pltpu.SemaphoreType.DMA,
    ],
    compiler_params=pltpu.CompilerParams(needs_layout_passes=False),
)
def _sc_edge_att(att_hbm, ei_hbm, out_hbm, tab, ibuf, obuf, insem, outsem, tabsem):
    w = lax.axis_index("s") * NC + lax.axis_index("c")
    base = (w * BASE_BLK + jnp.minimum(w, EXTRA)) * BLK

    def in_off(c):
        return pl.multiple_of(base + c * CC, BLK)

    pltpu.async_copy(att_hbm, tab, tabsem)
    for c0 in range(NBUF - 1):
        pltpu.async_copy(ei_hbm.at[:, pl.ds(in_off(c0), CC)], ibuf.at[c0], insem)
    pltpu.make_async_copy(att_hbm, tab, tabsem).wait()

    def group_body(g, carry):
        for b in range(NBUF):
            c = g * NBUF + b
            off = in_off(c)
            pltpu.make_async_copy(
                ei_hbm.at[:, pl.ds(off, CC)], ibuf.at[b], insem
            ).wait()

            @pl.when(c + NBUF - 1 < NCOPIES)
            def _():
                pltpu.async_copy(
                    ei_hbm.at[:, pl.ds(in_off(c + NBUF - 1), CC)],
                    ibuf.at[(b + NBUF - 1) % NBUF], insem,
                )

            @pl.when(c >= NBUF)
            def _():
                pltpu.make_async_copy(
                    obuf.at[b], out_hbm.at[pl.ds(off, CC)], outsem
                ).wait()

            @plsc.parallel_loop(0, CC // 16, 1, unroll=16)
            def _(j):
                s = ibuf[b, 0, pl.ds(j * 16, 16)]
                d = ibuf[b, 1, pl.ds(j * 16, 16)]
                obuf[b, pl.ds(j * 16, 16)] = (
                    plsc.load_gather(tab, [s]) * plsc.load_gather(tab, [d])
                )

            pltpu.async_copy(obuf.at[b], out_hbm.at[pl.ds(off, CC)], outsem)
        return carry

    lax.fori_loop(0, NCOPIES // NBUF, group_body, 0)

    for b in range(NBUF):
        pltpu.make_async_copy(
            obuf.at[b], out_hbm.at[pl.ds(in_off(0), CC)], outsem
        ).wait()

    ntail = 1 + (w < EXTRA).astype(jnp.int32)

    def tail_body(t, carry):
        toff = pl.multiple_of(base + NCOPIES * CC + t * BLK, BLK)
        pltpu.sync_copy(ei_hbm.at[:, pl.ds(toff, BLK)], ibuf.at[0, :, pl.ds(0, BLK)])

        @plsc.parallel_loop(0, BLK // 16, 1, unroll=8)
        def _(j):
            s = ibuf[0, 0, pl.ds(j * 16, 16)]
            d = ibuf[0, 1, pl.ds(j * 16, 16)]
            obuf[0, pl.ds(j * 16, 16)] = (
                plsc.load_gather(tab, [s]) * plsc.load_gather(tab, [d])
            )

        pltpu.sync_copy(obuf.at[0, pl.ds(0, BLK)], out_hbm.at[pl.ds(toff, BLK)])
        return carry

    lax.fori_loop(0, ntail, tail_body, 0)


def kernel(att_log_logits, edge_index, epoch):
    r = jnp.maximum(0.9 - (epoch // 10).astype(jnp.float32) * 0.1, 0.7)
    x = jnp.pad(att_log_logits.reshape(-1), (0, NPAD - N)).reshape(ROWS, LANES)

    att2d, loss = pl.pallas_call(
        _tc_att_loss,
        in_specs=[
            pl.BlockSpec(memory_space=pltpu.SMEM),
            pl.BlockSpec(memory_space=pltpu.VMEM),
        ],
        out_specs=[
            pl.BlockSpec(memory_space=pltpu.VMEM),
            pl.BlockSpec(memory_space=pltpu.SMEM),
        ],
        out_shape=[
            jax.ShapeDtypeStruct((ROWS, LANES), jnp.float32),
            jax.ShapeDtypeStruct((1,), jnp.float32),
        ],
    )(r.reshape(1), x)

    edge_att = _sc_edge_att(att2d.reshape(NPAD), edge_index)
    return edge_att.reshape(E, 1), loss[0]


# speedup vs baseline: 1693.9168x; 1.0261x over previous
import functools

import jax
import jax.numpy as jnp
from jax import lax
from jax.experimental import pallas as pl
from jax.experimental.pallas import tpu as pltpu
from jax.experimental.pallas import tpu_sc as plsc

N = 100000
E = 6400000
NPAD = 102400
ROWS = 800
TABW = 100096
LANES = 128

NC = 2
NS = 16
NW = NC * NS
BLK = 2048
NBLK = E // BLK
BASE_BLK = NBLK // NW
EXTRA = NBLK % NW
CC = BLK
NCOPIES = 95
NBUF = 5


def _tc_att_loss(r_ref, x_ref, att_ref, loss_ref):
    x = x_ref[...]
    att = jax.nn.sigmoid(x)
    att_ref[...] = att
    r = r_ref[0]
    row = lax.broadcasted_iota(jnp.int32, (ROWS, LANES), 0)
    col = lax.broadcasted_iota(jnp.int32, (ROWS, LANES), 1)
    valid = (row * LANES + col) < N
    term = (att * jnp.log(att / r + 1e-6)
            + (1.0 - att) * jnp.log((1.0 - att) / (1.0 - r + 1e-6) + 1e-6))
    loss_ref[0] = jnp.sum(jnp.where(valid, term, 0.0)) / N


_mesh = plsc.VectorSubcoreMesh(core_axis_name="c", subcore_axis_name="s")


@functools.partial(
    pl.kernel,
    mesh=_mesh,
    out_type=jax.ShapeDtypeStruct((E,), jnp.float32),
    scratch_types=[
        pltpu.VMEM((TABW,), jnp.float32),
        pltpu.VMEM((NBUF, 2, CC), jnp.int32),
        pltpu.VMEM((NBUF * CC,), jnp.float32),
        pltpu.SemaphoreType.DMA,
        pltpu.SemaphoreType.DMA,
        pltpu.SemaphoreType.DMA,
    ],
    compiler_params=pltpu.CompilerParams(needs_layout_passes=False),
)
def _sc_edge_att(att_hbm, ei_hbm, out_hbm, tab, ibuf, obuf, insem, outsem, tabsem):
    w = lax.axis_index("s") * NC + lax.axis_index("c")
    base = (w * BASE_BLK + jnp.minimum(w, EXTRA)) * BLK

    def in_off(c):
        return pl.multiple_of(base + c * CC, BLK)

    pltpu.async_copy(att_hbm.at[pl.ds(0, TABW)], tab, tabsem)
    for c0 in range(NBUF - 1):
        pltpu.async_copy(ei_hbm.at[:, pl.ds(in_off(c0), CC)], ibuf.at[c0], insem)
    pltpu.make_async_copy(att_hbm.at[pl.ds(0, TABW)], tab, tabsem).wait()

    def group_body(g, carry):
        for b in range(NBUF):
            c = g * NBUF + b
            off = in_off(c)
            pltpu.make_async_copy(
                ei_hbm.at[:, pl.ds(off, CC)], ibuf.at[b], insem
            ).wait()

            @pl.when(c + NBUF - 1 < NCOPIES)
            def _():
                pltpu.async_copy(
                    ei_hbm.at[:, pl.ds(in_off(c + NBUF - 1), CC)],
                    ibuf.at[(b + NBUF - 1) % NBUF], insem,
                )

            @pl.when(c >= NBUF)
            def _():
                pltpu.make_async_copy(
                    obuf.at[pl.ds(b * CC, CC)], out_hbm.at[pl.ds(off, CC)], outsem
                ).wait()

            @plsc.parallel_loop(0, CC // 16, 1, unroll=16)
            def _(j):
                s = ibuf[b, 0, pl.ds(j * 16, 16)]
                d = ibuf[b, 1, pl.ds(j * 16, 16)]
                obuf[pl.ds(b * CC + j * 16, 16)] = (
                    plsc.load_gather(tab, [s]) * plsc.load_gather(tab, [d])
                )

            pltpu.async_copy(obuf.at[pl.ds(b * CC, CC)], out_hbm.at[pl.ds(off, CC)], outsem)
        return carry

    lax.fori_loop(0, NCOPIES // NBUF, group_body, 0)

    for b in range(NBUF):
        pltpu.make_async_copy(
            obuf.at[pl.ds(b * CC, CC)], out_hbm.at[pl.ds(in_off(0), CC)], outsem
        ).wait()

    ntail = (BASE_BLK - NCOPIES) + (w < EXTRA).astype(jnp.int32)

    def tail_body(t, carry):
        toff = pl.multiple_of(base + NCOPIES * CC + t * BLK, BLK)
        pltpu.sync_copy(ei_hbm.at[:, pl.ds(toff, BLK)], ibuf.at[0, :, pl.ds(0, BLK)])

        @plsc.parallel_loop(0, BLK // 16, 1, unroll=8)
        def _(j):
            s = ibuf[0, 0, pl.ds(j * 16, 16)]
            d = ibuf[0, 1, pl.ds(j * 16, 16)]
            obuf[pl.ds(j * 16, 16)] = (
                plsc.load_gather(tab, [s]) * plsc.load_gather(tab, [d])
            )

        pltpu.sync_copy(obuf.at[pl.ds(0, BLK)], out_hbm.at[pl.ds(toff, BLK)])
        return carry

    lax.fori_loop(0, ntail, tail_body, 0)


def kernel(att_log_logits, edge_index, epoch):
    r = jnp.maximum(0.9 - (epoch // 10).astype(jnp.float32) * 0.1, 0.7)
    x = jnp.pad(att_log_logits.reshape(-1), (0, NPAD - N)).reshape(ROWS, LANES)

    att2d, loss = pl.pallas_call(
        _tc_att_loss,
        in_specs=[
            pl.BlockSpec(memory_space=pltpu.SMEM),
            pl.BlockSpec(memory_space=pltpu.VMEM),
        ],
        out_specs=[
            pl.BlockSpec(memory_space=pltpu.VMEM),
            pl.BlockSpec(memory_space=pltpu.SMEM),
        ],
        out_shape=[
            jax.ShapeDtypeStruct((ROWS, LANES), jnp.float32),
            jax.ShapeDtypeStruct((1,), jnp.float32),
        ],
    )(r.reshape(1), x)

    edge_att = _sc_edge_att(att2d.reshape(NPAD), edge_index)
    return edge_att.reshape(E, 1), loss[0]


# speedup vs baseline: 1759.8396x; 1.0389x over previous
import functools

import jax
import jax.numpy as jnp
from jax import lax
from jax.experimental import pallas as pl
from jax.experimental.pallas import tpu as pltpu
from jax.experimental.pallas import tpu_sc as plsc

N = 100000
E = 6400000
NPAD = 102400
ROWS = 800
TABW = 100096
LANES = 128

NC = 2
NS = 16
NW = NC * NS
BLK = 2048
NBLK = E // BLK
BASE_BLK = NBLK // NW
EXTRA = NBLK % NW
CC = BLK
NCOPIES = 96
NBUF = 4
SHROW = 6272


def _tc_att_loss(r_ref, x_ref, att_ref, loss_ref):
    x = x_ref[...]
    att = jax.nn.sigmoid(x)
    att_ref[...] = att
    r = r_ref[0]
    row = lax.broadcasted_iota(jnp.int32, (ROWS, LANES), 0)
    col = lax.broadcasted_iota(jnp.int32, (ROWS, LANES), 1)
    valid = (row * LANES + col) < N
    term = (att * jnp.log(att / r + 1e-6)
            + (1.0 - att) * jnp.log((1.0 - att) / (1.0 - r + 1e-6) + 1e-6))
    loss_ref[0] = jnp.sum(jnp.where(valid, term, 0.0)) / N


_mesh = plsc.VectorSubcoreMesh(core_axis_name="c", subcore_axis_name="s")


@functools.partial(
    pl.kernel,
    mesh=_mesh,
    out_type=jax.ShapeDtypeStruct((E,), jnp.float32),
    scratch_types=[
        pltpu.VMEM((TABW,), jnp.float32),
        pltpu.VMEM((NBUF, 2, CC), jnp.int32),
        pltpu.VMEM((NBUF * CC,), jnp.float32),
        pltpu.VMEM_SHARED((NS, SHROW), jnp.float32),
        pltpu.SemaphoreType.DMA,
        pltpu.SemaphoreType.DMA,
        pltpu.SemaphoreType.DMA,
    ],
    compiler_params=pltpu.CompilerParams(needs_layout_passes=False),
)
def _sc_edge_att(att_hbm, ei_hbm, out_hbm, tab, ibuf, obuf, shtab, insem, outsem, tabsem):
    w = lax.axis_index("s") * NC + lax.axis_index("c")
    base = (w * BASE_BLK + jnp.minimum(w, EXTRA)) * BLK

    def in_off(c):
        return pl.multiple_of(base + c * CC, BLK)

    s_id = lax.axis_index("s")
    sh_off = pl.multiple_of(s_id * SHROW, 128)
    pltpu.async_copy(att_hbm.at[pl.ds(sh_off, SHROW)], shtab.at[s_id], tabsem)
    for c0 in range(NBUF - 1):
        pltpu.async_copy(ei_hbm.at[:, pl.ds(in_off(c0), CC)], ibuf.at[c0], insem)
    pltpu.make_async_copy(att_hbm.at[pl.ds(sh_off, SHROW)], shtab.at[s_id], tabsem).wait()
    plsc.subcore_barrier()
    for k in range(NS - 1):
        pltpu.async_copy(shtab.at[k], tab.at[pl.ds(k * SHROW, SHROW)], tabsem)
    _last = TABW - (NS - 1) * SHROW
    pltpu.async_copy(
        shtab.at[NS - 1, pl.ds(0, _last)],
        tab.at[pl.ds((NS - 1) * SHROW, _last)], tabsem,
    )
    for k in range(NS - 1):
        pltpu.make_async_copy(shtab.at[k], tab.at[pl.ds(k * SHROW, SHROW)], tabsem).wait()
    pltpu.make_async_copy(
        shtab.at[NS - 1, pl.ds(0, _last)],
        tab.at[pl.ds((NS - 1) * SHROW, _last)], tabsem,
    ).wait()

    def group_body(g, carry):
        for b in range(NBUF):
            c = g * NBUF + b
            off = in_off(c)
            pltpu.make_async_copy(
                ei_hbm.at[:, pl.ds(off, CC)], ibuf.at[b], insem
            ).wait()

            @pl.when(c + NBUF - 1 < NCOPIES)
            def _():
                pltpu.async_copy(
                    ei_hbm.at[:, pl.ds(in_off(c + NBUF - 1), CC)],
                    ibuf.at[(b + NBUF - 1) % NBUF], insem,
                )

            @pl.when(c >= NBUF)
            def _():
                pltpu.make_async_copy(
                    obuf.at[pl.ds(b * CC, CC)], out_hbm.at[pl.ds(off, CC)], outsem
                ).wait()

            @plsc.parallel_loop(0, CC // 16, 1, unroll=16)
            def _(j):
                s = ibuf[b, 0, pl.ds(j * 16, 16)]
                d = ibuf[b, 1, pl.ds(j * 16, 16)]
                obuf[pl.ds(b * CC + j * 16, 16)] = (
                    plsc.load_gather(tab, [s]) * plsc.load_gather(tab, [d])
                )

            pltpu.async_copy(obuf.at[pl.ds(b * CC, CC)], out_hbm.at[pl.ds(off, CC)], outsem)
        return carry

    lax.fori_loop(0, NCOPIES // NBUF, group_body, 0)

    for b in range(NBUF):
        pltpu.make_async_copy(
            obuf.at[pl.ds(b * CC, CC)], out_hbm.at[pl.ds(in_off(0), CC)], outsem
        ).wait()

    ntail = (BASE_BLK - NCOPIES) + (w < EXTRA).astype(jnp.int32)

    def tail_body(t, carry):
        toff = pl.multiple_of(base + NCOPIES * CC + t * BLK, BLK)
        pltpu.sync_copy(ei_hbm.at[:, pl.ds(toff, BLK)], ibuf.at[0, :, pl.ds(0, BLK)])

        @plsc.parallel_loop(0, BLK // 16, 1, unroll=8)
        def _(j):
            s = ibuf[0, 0, pl.ds(j * 16, 16)]
            d = ibuf[0, 1, pl.ds(j * 16, 16)]
            obuf[pl.ds(j * 16, 16)] = (
                plsc.load_gather(tab, [s]) * plsc.load_gather(tab, [d])
            )

        pltpu.sync_copy(obuf.at[pl.ds(0, BLK)], out_hbm.at[pl.ds(toff, BLK)])
        return carry

    lax.fori_loop(0, ntail, tail_body, 0)


def kernel(att_log_logits, edge_index, epoch):
    r = jnp.maximum(0.9 - (epoch // 10).astype(jnp.float32) * 0.1, 0.7)
    x = jnp.pad(att_log_logits.reshape(-1), (0, NPAD - N)).reshape(ROWS, LANES)

    att2d, loss = pl.pallas_call(
        _tc_att_loss,
        in_specs=[
            pl.BlockSpec(memory_space=pltpu.SMEM),
            pl.BlockSpec(memory_space=pltpu.VMEM),
        ],
        out_specs=[
            pl.BlockSpec(memory_space=pltpu.VMEM),
            pl.BlockSpec(memory_space=pltpu.SMEM),
        ],
        out_shape=[
            jax.ShapeDtypeStruct((ROWS, LANES), jnp.float32),
            jax.ShapeDtypeStruct((1,), jnp.float32),
        ],
    )(r.reshape(1), x)

    edge_att = _sc_edge_att(att2d.reshape(NPAD), edge_index)
    return edge_att.reshape(E, 1), loss[0]


# speedup vs baseline: 1763.0684x; 1.0018x over previous
import functools

import jax
import jax.numpy as jnp
from jax import lax
from jax.experimental import pallas as pl
from jax.experimental.pallas import tpu as pltpu
from jax.experimental.pallas import tpu_sc as plsc

N = 100000
E = 6400000
NPAD = 102400
ROWS = 800
TABW = 100096
LANES = 128

NC = 2
NS = 16
NW = NC * NS
BLK = 2048
NBLK = E // BLK
BASE_BLK = NBLK // NW
EXTRA = NBLK % NW
CC = BLK
NCOPIES = 96
NBUF = 4
SHROW = 6272


def _tc_att_loss(r_ref, x_ref, att_ref, loss_ref):
    x = x_ref[...]
    att = jax.nn.sigmoid(x)
    att_ref[...] = att
    r = r_ref[0]
    row = lax.broadcasted_iota(jnp.int32, (ROWS, LANES), 0)
    col = lax.broadcasted_iota(jnp.int32, (ROWS, LANES), 1)
    valid = (row * LANES + col) < N
    term = (att * jnp.log(att / r + 1e-6)
            + (1.0 - att) * jnp.log((1.0 - att) / (1.0 - r + 1e-6) + 1e-6))
    loss_ref[0] = jnp.sum(jnp.where(valid, term, 0.0)) / N


_mesh = plsc.VectorSubcoreMesh(core_axis_name="c", subcore_axis_name="s")


@functools.partial(
    pl.kernel,
    mesh=_mesh,
    out_type=jax.ShapeDtypeStruct((E,), jnp.float32),
    scratch_types=[
        pltpu.VMEM((TABW,), jnp.float32),
        pltpu.VMEM((NBUF, 2, CC), jnp.int32),
        pltpu.VMEM((NBUF * CC,), jnp.float32),
        pltpu.VMEM_SHARED((NS, SHROW), jnp.float32),
        pltpu.SemaphoreType.DMA,
        pltpu.SemaphoreType.DMA,
        pltpu.SemaphoreType.DMA,
    ],
    compiler_params=pltpu.CompilerParams(
        needs_layout_passes=False,
        disable_bounds_checks=True,
        disable_semaphore_checks=True,
    ),
)
def _sc_edge_att(att_hbm, ei_hbm, out_hbm, tab, ibuf, obuf, shtab, insem, outsem, tabsem):
    w = lax.axis_index("s") * NC + lax.axis_index("c")
    base = (w * BASE_BLK + jnp.minimum(w, EXTRA)) * BLK

    def in_off(c):
        return pl.multiple_of(base + c * CC, BLK)

    s_id = lax.axis_index("s")
    sh_off = pl.multiple_of(s_id * SHROW, 128)
    pltpu.async_copy(att_hbm.at[pl.ds(sh_off, SHROW)], shtab.at[s_id], tabsem)
    for c0 in range(NBUF - 1):
        pltpu.async_copy(ei_hbm.at[:, pl.ds(in_off(c0), CC)], ibuf.at[c0], insem)
    pltpu.make_async_copy(att_hbm.at[pl.ds(sh_off, SHROW)], shtab.at[s_id], tabsem).wait()
    plsc.subcore_barrier()
    for k in range(NS - 1):
        pltpu.async_copy(shtab.at[k], tab.at[pl.ds(k * SHROW, SHROW)], tabsem)
    _last = TABW - (NS - 1) * SHROW
    pltpu.async_copy(
        shtab.at[NS - 1, pl.ds(0, _last)],
        tab.at[pl.ds((NS - 1) * SHROW, _last)], tabsem,
    )
    for k in range(NS - 1):
        pltpu.make_async_copy(shtab.at[k], tab.at[pl.ds(k * SHROW, SHROW)], tabsem).wait()
    pltpu.make_async_copy(
        shtab.at[NS - 1, pl.ds(0, _last)],
        tab.at[pl.ds((NS - 1) * SHROW, _last)], tabsem,
    ).wait()

    def group_body(g, carry):
        for b in range(NBUF):
            c = g * NBUF + b
            off = in_off(c)
            pltpu.make_async_copy(
                ei_hbm.at[:, pl.ds(off, CC)], ibuf.at[b], insem
            ).wait()

            @pl.when(c + NBUF - 1 < NCOPIES)
            def _():
                pltpu.async_copy(
                    ei_hbm.at[:, pl.ds(in_off(c + NBUF - 1), CC)],
                    ibuf.at[(b + NBUF - 1) % NBUF], insem,
                )

            @pl.when(c >= NBUF)
            def _():
                pltpu.make_async_copy(
                    obuf.at[pl.ds(b * CC, CC)], out_hbm.at[pl.ds(off, CC)], outsem
                ).wait()

            @plsc.parallel_loop(0, CC // 16, 1, unroll=16)
            def _(j):
                s = ibuf[b, 0, pl.ds(j * 16, 16)]
                d = ibuf[b, 1, pl.ds(j * 16, 16)]
                obuf[pl.ds(b * CC + j * 16, 16)] = (
                    plsc.load_gather(tab, [s]) * plsc.load_gather(tab, [d])
                )

            pltpu.async_copy(obuf.at[pl.ds(b * CC, CC)], out_hbm.at[pl.ds(off, CC)], outsem)
        return carry

    lax.fori_loop(0, NCOPIES // NBUF, group_body, 0)

    for b in range(NBUF):
        pltpu.make_async_copy(
            obuf.at[pl.ds(b * CC, CC)], out_hbm.at[pl.ds(in_off(0), CC)], outsem
        ).wait()

    ntail = (BASE_BLK - NCOPIES) + (w < EXTRA).astype(jnp.int32)

    def tail_body(t, carry):
        toff = pl.multiple_of(base + NCOPIES * CC + t * BLK, BLK)
        pltpu.sync_copy(ei_hbm.at[:, pl.ds(toff, BLK)], ibuf.at[0, :, pl.ds(0, BLK)])

        @plsc.parallel_loop(0, BLK // 16, 1, unroll=8)
        def _(j):
            s = ibuf[0, 0, pl.ds(j * 16, 16)]
            d = ibuf[0, 1, pl.ds(j * 16, 16)]
            obuf[pl.ds(j * 16, 16)] = (
                plsc.load_gather(tab, [s]) * plsc.load_gather(tab, [d])
            )

        pltpu.sync_copy(obuf.at[pl.ds(0, BLK)], out_hbm.at[pl.ds(toff, BLK)])
        return carry

    lax.fori_loop(0, ntail, tail_body, 0)


def kernel(att_log_logits, edge_index, epoch):
    r = jnp.maximum(0.9 - (epoch // 10).astype(jnp.float32) * 0.1, 0.7)
    x = jnp.pad(att_log_logits.reshape(-1), (0, NPAD - N)).reshape(ROWS, LANES)

    att2d, loss = pl.pallas_call(
        _tc_att_loss,
        in_specs=[
            pl.BlockSpec(memory_space=pltpu.SMEM),
            pl.BlockSpec(memory_space=pltpu.VMEM),
        ],
        out_specs=[
            pl.BlockSpec(memory_space=pltpu.VMEM),
            pl.BlockSpec(memory_space=pltpu.SMEM),
        ],
        out_shape=[
            jax.ShapeDtypeStruct((ROWS, LANES), jnp.float32),
            jax.ShapeDtypeStruct((1,), jnp.float32),
        ],
    )(r.reshape(1), x)

    edge_att = _sc_edge_att(att2d.reshape(NPAD), edge_index)
    return edge_att.reshape(E, 1), loss[0]
